# baseline (device time: 67980 ns/iter reference)
import jax
import jax.numpy as jnp
from jax import lax
from jax.experimental import pallas as pl
from jax.experimental.pallas import tpu as pltpu

N_DEV = 16
B, SQ, SKV, DM = 2, 512, 512, 768
DH = 64
H_LOC = 8
DLOC = H_LOC * DH
CH = SQ // N_DEV
HDM = DM // 2
HS = SQ // 2

MASKS_A = [1, 4, 2, 8]
MASKS_B = [4, 1, 8, 2]
_BITPOS = {1: 0, 2: 1, 4: 2, 8: 3}


def _perm(masks, c):
    pos = 0
    for k, m in enumerate(masks):
        pos |= ((c >> _BITPOS[m]) & 1) << (3 - k)
    return pos


def kernel(x, Wq, K_ext, V_ext, Wo):
    my = lax.axis_index("i")
    Wq_l = lax.dynamic_slice(Wq, (0, my * DLOC), (DM, DLOC))
    Wo_l = lax.dynamic_slice(Wo, (my * DLOC, 0), (DLOC, DM))

    def body(x_ref, wq_ref, k_ref, v_ref, wo_ref, out_ref,
             acc_a, acc_b, st_a, st_b,
             rsa_s, rsa_r, rsb_s, rsb_r, aga_s, aga_r, agb_s, agb_r):
        my_i = lax.axis_index("i")

        qb = lax.broadcasted_iota(jnp.int32, (SQ, SKV), 0) // 64
        kb = lax.broadcasted_iota(jnp.int32, (SQ, SKV), 1) // 64
        mask = kb <= qb
        mask_top = mask[:HS, :HS]
        mask_bot = mask[HS:, :]
        wq = wq_ref[...].astype(jnp.bfloat16)
        wo = wo_ref[...].astype(jnp.bfloat16)
        for b in range(B):
            xb = x_ref[b].astype(jnp.bfloat16)
            q16 = jnp.dot(xb, wq,
                          preferred_element_type=jnp.float32).astype(jnp.bfloat16)
            top_parts = []
            bot_parts = []
            for h in range(H_LOC):
                qh = q16[:, h * DH:(h + 1) * DH]
                kh = k_ref[b, :, h, :].astype(jnp.bfloat16)
                vh = v_ref[b, :, h, :].astype(jnp.bfloat16)
                s_t = lax.dot_general(
                    qh[:HS], kh[:HS], (((1,), (1,)), ((), ())),
                    preferred_element_type=jnp.float32) * 0.125
                w_t = jnp.exp(jnp.where(mask_top, s_t, -1e9))
                sum_t = jnp.sum(w_t, axis=1, keepdims=True)
                ctx_t = jnp.dot(w_t.astype(jnp.bfloat16), vh[:HS],
                                preferred_element_type=jnp.float32)
                top_parts.append(ctx_t / sum_t)
                s_b = lax.dot_general(
                    qh[HS:], kh, (((1,), (1,)), ((), ())),
                    preferred_element_type=jnp.float32) * 0.125
                w_b = jnp.exp(jnp.where(mask_bot, s_b, -1e9))
                sum_b = jnp.sum(w_b, axis=1, keepdims=True)
                ctx_b = jnp.dot(w_b.astype(jnp.bfloat16), vh,
                                preferred_element_type=jnp.float32)
                bot_parts.append(ctx_b / sum_b)
            ctx = jnp.concatenate(
                [jnp.concatenate(top_parts, axis=1),
                 jnp.concatenate(bot_parts, axis=1)],
                axis=0).astype(jnp.bfloat16)
            partial = jnp.dot(ctx, wo, preferred_element_type=jnp.float32)
            p16 = partial.astype(jnp.bfloat16)
            for c in range(N_DEV):
                acc_a[_perm(MASKS_A, c), b] = p16[c * CH:(c + 1) * CH, :HDM]
                acc_b[_perm(MASKS_B, c), b] = p16[c * CH:(c + 1) * CH, HDM:]

        bits_a = [jnp.bitwise_and(lax.shift_right_logical(my_i, _BITPOS[m]), 1)
                  for m in MASKS_A]
        bits_b = [jnp.bitwise_and(lax.shift_right_logical(my_i, _BITPOS[m]), 1)
                  for m in MASKS_B]
        S_a = 0
        S_b = 0
        o = 0
        for k in range(4):
            h = 8 >> k
            send_a = S_a + (1 - bits_a[k]) * h
            rdma_a = pltpu.make_async_remote_copy(
                src_ref=acc_a.at[pl.ds(send_a, h)],
                dst_ref=st_a.at[pl.ds(o, h)],
                send_sem=rsa_s.at[k], recv_sem=rsa_r.at[k],
                device_id=(jnp.bitwise_xor(my_i, MASKS_A[k]),),
                device_id_type=pl.DeviceIdType.MESH,
            )
            rdma_a.start()
            send_b = S_b + (1 - bits_b[k]) * h
            rdma_b = pltpu.make_async_remote_copy(
                src_ref=acc_b.at[pl.ds(send_b, h)],
                dst_ref=st_b.at[pl.ds(o, h)],
                send_sem=rsb_s.at[k], recv_sem=rsb_r.at[k],
                device_id=(jnp.bitwise_xor(my_i, MASKS_B[k]),),
                device_id_type=pl.DeviceIdType.MESH,
            )
            rdma_b.start()
            rdma_a.wait()
            S_a = S_a + bits_a[k] * h
            acc_a[pl.ds(S_a, h)] = (
                acc_a[pl.ds(S_a, h)].astype(jnp.float32)
                + st_a[pl.ds(o, h)].astype(jnp.float32)).astype(jnp.bfloat16)
            rdma_b.wait()
            S_b = S_b + bits_b[k] * h
            acc_b[pl.ds(S_b, h)] = (
                acc_b[pl.ds(S_b, h)].astype(jnp.float32)
                + st_b[pl.ds(o, h)].astype(jnp.float32)).astype(jnp.bfloat16)
            o += h

        T_a = S_a
        T_b = S_b
        for k in range(4):
            g = 1 << k
            rdma_a = pltpu.make_async_remote_copy(
                src_ref=acc_a.at[pl.ds(T_a, g)],
                dst_ref=acc_a.at[pl.ds(T_a, g)],
                send_sem=aga_s.at[k], recv_sem=aga_r.at[k],
                device_id=(jnp.bitwise_xor(my_i, MASKS_A[3 - k]),),
                device_id_type=pl.DeviceIdType.MESH,
            )
            rdma_a.start()
            rdma_b = pltpu.make_async_remote_copy(
                src_ref=acc_b.at[pl.ds(T_b, g)],
                dst_ref=acc_b.at[pl.ds(T_b, g)],
                send_sem=agb_s.at[k], recv_sem=agb_r.at[k],
                device_id=(jnp.bitwise_xor(my_i, MASKS_B[3 - k]),),
                device_id_type=pl.DeviceIdType.MESH,
            )
            rdma_b.start()
            rdma_a.wait()
            sib = jnp.left_shift(
                jnp.bitwise_xor(lax.shift_right_logical(T_a, k), 1), k)
            T_a = jnp.minimum(T_a, sib)
            rdma_b.wait()
            sib = jnp.left_shift(
                jnp.bitwise_xor(lax.shift_right_logical(T_b, k), 1), k)
            T_b = jnp.minimum(T_b, sib)

        for c in range(N_DEV):
            sl = slice(c * CH, (c + 1) * CH)
            out_ref[:, sl, :HDM] = acc_a[_perm(MASKS_A, c)].astype(jnp.float32)
            out_ref[:, sl, HDM:] = acc_b[_perm(MASKS_B, c)].astype(jnp.float32)

    return pl.pallas_call(
        body,
        out_shape=jax.ShapeDtypeStruct((B, SQ, DM), jnp.float32),
        in_specs=[pl.BlockSpec(memory_space=pltpu.VMEM)] * 5,
        out_specs=pl.BlockSpec(memory_space=pltpu.VMEM),
        scratch_shapes=[
            pltpu.VMEM((N_DEV, B, CH, HDM), jnp.bfloat16),
            pltpu.VMEM((N_DEV, B, CH, HDM), jnp.bfloat16),
            pltpu.VMEM((15, B, CH, HDM), jnp.bfloat16),
            pltpu.VMEM((15, B, CH, HDM), jnp.bfloat16),
            pltpu.SemaphoreType.DMA((4,)),
            pltpu.SemaphoreType.DMA((4,)),
            pltpu.SemaphoreType.DMA((4,)),
            pltpu.SemaphoreType.DMA((4,)),
            pltpu.SemaphoreType.DMA((4,)),
            pltpu.SemaphoreType.DMA((4,)),
            pltpu.SemaphoreType.DMA((4,)),
            pltpu.SemaphoreType.DMA((4,)),
        ],
    )(x, Wq_l, K_ext, V_ext, Wo_l)


# device time: 63744 ns/iter; 1.0665x vs baseline; 1.0665x over previous
import jax
import jax.numpy as jnp
from jax import lax
from jax.experimental import pallas as pl
from jax.experimental.pallas import tpu as pltpu

N_DEV = 16
B, SQ, SKV, DM = 2, 512, 512, 768
DH = 64
H_LOC = 8
DLOC = H_LOC * DH
CH = SQ // N_DEV
HDM = DM // 2
HS = SQ // 2

MASKS_A = [1, 4, 2, 8]
MASKS_B = [4, 1, 8, 2]


def _parity(v: int) -> int:
    return bin(v).count("1") & 1


def _perm(masks, c):
    pos = 0
    for k, m in enumerate(masks):
        pos |= _parity(c & m) << (3 - k)
    return pos


def _tparity(v, m):
    p = jnp.bitwise_and(v, m)
    p = jnp.bitwise_xor(p, lax.shift_right_logical(p, 2))
    p = jnp.bitwise_xor(p, lax.shift_right_logical(p, 1))
    return jnp.bitwise_and(p, 1)


def kernel(x, Wq, K_ext, V_ext, Wo):
    my = lax.axis_index("i")
    Wq_l = lax.dynamic_slice(Wq, (0, my * DLOC), (DM, DLOC))
    Wo_l = lax.dynamic_slice(Wo, (my * DLOC, 0), (DLOC, DM))

    def body(x_ref, wq_ref, k_ref, v_ref, wo_ref, out_ref,
             acc_a, acc_b, st_a, st_b,
             rsa_s, rsa_r, rsb_s, rsb_r, aga_s, aga_r, agb_s, agb_r):
        my_i = lax.axis_index("i")
        bits_a = [_tparity(my_i, m) for m in MASKS_A]
        bits_b = [_tparity(my_i, m) for m in MASKS_B]
        send0_a = (1 - bits_a[0]) * 8
        send0_b = (1 - bits_b[0]) * 8

        qb = lax.broadcasted_iota(jnp.int32, (SQ, SKV), 0) // 64
        kb = lax.broadcasted_iota(jnp.int32, (SQ, SKV), 1) // 64
        mask = kb <= qb
        mask_top = mask[:HS, :HS]
        mask_bot = mask[HS:, :]
        wq = wq_ref[...].astype(jnp.bfloat16)
        wo = wo_ref[...].astype(jnp.bfloat16)
        r0 = [None, None, None, None]
        for b in range(B):
            xb = x_ref[b].astype(jnp.bfloat16)
            q16 = jnp.dot(xb, wq,
                          preferred_element_type=jnp.float32).astype(jnp.bfloat16)
            top_parts = []
            bot_parts = []
            for h in range(H_LOC):
                qh = q16[:, h * DH:(h + 1) * DH]
                kh = k_ref[b, :, h, :].astype(jnp.bfloat16)
                vh = v_ref[b, :, h, :].astype(jnp.bfloat16)
                s_t = lax.dot_general(
                    qh[:HS], kh[:HS], (((1,), (1,)), ((), ())),
                    preferred_element_type=jnp.float32) * 0.125
                w_t = jnp.exp(jnp.where(mask_top, s_t, -1e9))
                sum_t = jnp.sum(w_t, axis=1, keepdims=True)
                ctx_t = jnp.dot(w_t.astype(jnp.bfloat16), vh[:HS],
                                preferred_element_type=jnp.float32)
                top_parts.append(ctx_t / sum_t)
                s_b = lax.dot_general(
                    qh[HS:], kh, (((1,), (1,)), ((), ())),
                    preferred_element_type=jnp.float32) * 0.125
                w_b = jnp.exp(jnp.where(mask_bot, s_b, -1e9))
                sum_b = jnp.sum(w_b, axis=1, keepdims=True)
                ctx_b = jnp.dot(w_b.astype(jnp.bfloat16), vh,
                                preferred_element_type=jnp.float32)
                bot_parts.append(ctx_b / sum_b)
            ctx = jnp.concatenate(
                [jnp.concatenate(top_parts, axis=1),
                 jnp.concatenate(bot_parts, axis=1)],
                axis=0).astype(jnp.bfloat16)
            partial = jnp.dot(ctx, wo, preferred_element_type=jnp.float32)
            p16 = partial.astype(jnp.bfloat16)
            for c in range(N_DEV):
                acc_a[_perm(MASKS_A, c), b] = p16[c * CH:(c + 1) * CH, :HDM]
                acc_b[_perm(MASKS_B, c), b] = p16[c * CH:(c + 1) * CH, HDM:]
            rd_a = pltpu.make_async_remote_copy(
                src_ref=acc_a.at[pl.ds(send0_a, 8), b],
                dst_ref=st_a.at[pl.ds(0, 8), b],
                send_sem=rsa_s.at[2 * b], recv_sem=rsa_r.at[2 * b],
                device_id=(jnp.bitwise_xor(my_i, MASKS_A[0]),),
                device_id_type=pl.DeviceIdType.MESH,
            )
            rd_a.start()
            rd_b = pltpu.make_async_remote_copy(
                src_ref=acc_b.at[pl.ds(send0_b, 8), b],
                dst_ref=st_b.at[pl.ds(0, 8), b],
                send_sem=rsb_s.at[2 * b], recv_sem=rsb_r.at[2 * b],
                device_id=(jnp.bitwise_xor(my_i, MASKS_B[0]),),
                device_id_type=pl.DeviceIdType.MESH,
            )
            rd_b.start()
            r0[2 * b], r0[2 * b + 1] = rd_a, rd_b

        for rd in r0:
            rd.wait()
        S_a = bits_a[0] * 8
        S_b = bits_b[0] * 8
        acc_a[pl.ds(S_a, 8)] = (
            acc_a[pl.ds(S_a, 8)].astype(jnp.float32)
            + st_a[pl.ds(0, 8)].astype(jnp.float32)).astype(jnp.bfloat16)
        acc_b[pl.ds(S_b, 8)] = (
            acc_b[pl.ds(S_b, 8)].astype(jnp.float32)
            + st_b[pl.ds(0, 8)].astype(jnp.float32)).astype(jnp.bfloat16)

        o = 8
        _SEMK = {1: 1, 2: 3, 3: 4}
        for k in (1, 2, 3):
            if k == 1:
                src_a = S_a + (1 - bits_a[1]) * 4
                src_b = S_b + (1 - bits_b[1]) * 4
            else:
                src_a, src_b = S_a, S_b
            rd_a = pltpu.make_async_remote_copy(
                src_ref=acc_a.at[pl.ds(src_a, 4)],
                dst_ref=st_a.at[pl.ds(o, 4)],
                send_sem=rsa_s.at[_SEMK[k]], recv_sem=rsa_r.at[_SEMK[k]],
                device_id=(jnp.bitwise_xor(my_i, MASKS_A[k]),),
                device_id_type=pl.DeviceIdType.MESH,
            )
            rd_a.start()
            rd_b = pltpu.make_async_remote_copy(
                src_ref=acc_b.at[pl.ds(src_b, 4)],
                dst_ref=st_b.at[pl.ds(o, 4)],
                send_sem=rsb_s.at[_SEMK[k]], recv_sem=rsb_r.at[_SEMK[k]],
                device_id=(jnp.bitwise_xor(my_i, MASKS_B[k]),),
                device_id_type=pl.DeviceIdType.MESH,
            )
            rd_b.start()
            rd_a.wait()
            if k == 1:
                S_a = S_a + bits_a[1] * 4
            acc_a[pl.ds(S_a, 4)] = (
                acc_a[pl.ds(S_a, 4)].astype(jnp.float32)
                + st_a[pl.ds(o, 4)].astype(jnp.float32)).astype(jnp.bfloat16)
            rd_b.wait()
            if k == 1:
                S_b = S_b + bits_b[1] * 4
            acc_b[pl.ds(S_b, 4)] = (
                acc_b[pl.ds(S_b, 4)].astype(jnp.float32)
                + st_b[pl.ds(o, 4)].astype(jnp.float32)).astype(jnp.bfloat16)
            o += 4

        for k in range(2):
            g = 4 << k
            m_a = MASKS_A[1 - k]
            m_b = MASKS_B[1 - k]
            rd_a = pltpu.make_async_remote_copy(
                src_ref=acc_a.at[pl.ds(S_a, g)],
                dst_ref=acc_a.at[pl.ds(S_a, g)],
                send_sem=aga_s.at[k], recv_sem=aga_r.at[k],
                device_id=(jnp.bitwise_xor(my_i, m_a),),
                device_id_type=pl.DeviceIdType.MESH,
            )
            rd_a.start()
            rd_b = pltpu.make_async_remote_copy(
                src_ref=acc_b.at[pl.ds(S_b, g)],
                dst_ref=acc_b.at[pl.ds(S_b, g)],
                send_sem=agb_s.at[k], recv_sem=agb_r.at[k],
                device_id=(jnp.bitwise_xor(my_i, m_b),),
                device_id_type=pl.DeviceIdType.MESH,
            )
            rd_b.start()
            rd_a.wait()
            shift = 2 + k
            sib = jnp.left_shift(jnp.bitwise_xor(
                lax.shift_right_logical(S_a, shift), 1), shift)
            S_a = jnp.minimum(S_a, sib)
            rd_b.wait()
            sib = jnp.left_shift(jnp.bitwise_xor(
                lax.shift_right_logical(S_b, shift), 1), shift)
            S_b = jnp.minimum(S_b, sib)

        for c in range(N_DEV):
            sl = slice(c * CH, (c + 1) * CH)
            out_ref[:, sl, :HDM] = acc_a[_perm(MASKS_A, c)].astype(jnp.float32)
            out_ref[:, sl, HDM:] = acc_b[_perm(MASKS_B, c)].astype(jnp.float32)

    return pl.pallas_call(
        body,
        out_shape=jax.ShapeDtypeStruct((B, SQ, DM), jnp.float32),
        in_specs=[pl.BlockSpec(memory_space=pltpu.VMEM)] * 5,
        out_specs=pl.BlockSpec(memory_space=pltpu.VMEM),
        scratch_shapes=[
            pltpu.VMEM((N_DEV, B, CH, HDM), jnp.bfloat16),
            pltpu.VMEM((N_DEV, B, CH, HDM), jnp.bfloat16),
            pltpu.VMEM((20, B, CH, HDM), jnp.bfloat16),
            pltpu.VMEM((20, B, CH, HDM), jnp.bfloat16),
            pltpu.SemaphoreType.DMA((5,)),
            pltpu.SemaphoreType.DMA((5,)),
            pltpu.SemaphoreType.DMA((5,)),
            pltpu.SemaphoreType.DMA((5,)),
            pltpu.SemaphoreType.DMA((2,)),
            pltpu.SemaphoreType.DMA((2,)),
            pltpu.SemaphoreType.DMA((2,)),
            pltpu.SemaphoreType.DMA((2,)),
        ],
    )(x, Wq_l, K_ext, V_ext, Wo_l)


# device time: 63468 ns/iter; 1.0711x vs baseline; 1.0043x over previous
import jax
import jax.numpy as jnp
from jax import lax
from jax.experimental import pallas as pl
from jax.experimental.pallas import tpu as pltpu

N_DEV = 16
B, SQ, SKV, DM = 2, 512, 512, 768
DH = 64
H_LOC = 8
DLOC = H_LOC * DH
CH = SQ // N_DEV
HDM = DM // 2
HS = SQ // 2

MASKS_A = [1, 4, 2, 8]
MASKS_B = [4, 1, 8, 2]


def _parity(v: int) -> int:
    return bin(v).count("1") & 1


def _perm(masks, c):
    pos = 0
    for k, m in enumerate(masks):
        pos |= _parity(c & m) << (3 - k)
    return pos


def _tparity(v, m):
    p = jnp.bitwise_and(v, m)
    p = jnp.bitwise_xor(p, lax.shift_right_logical(p, 2))
    p = jnp.bitwise_xor(p, lax.shift_right_logical(p, 1))
    return jnp.bitwise_and(p, 1)


def kernel(x, Wq, K_ext, V_ext, Wo):
    my = lax.axis_index("i")
    Wq_l = lax.dynamic_slice(Wq, (0, my * DLOC), (DM, DLOC))
    Wo_l = lax.dynamic_slice(Wo, (my * DLOC, 0), (DLOC, DM))

    def body(x_ref, wq_ref, k_ref, v_ref, wo_ref, out_ref,
             acc_a, acc_b, st_a, st_b,
             rsa_s, rsa_r, rsb_s, rsb_r, aga_s, aga_r, agb_s, agb_r):
        my_i = lax.axis_index("i")
        bits_a = [_tparity(my_i, m) for m in MASKS_A]
        bits_b = [_tparity(my_i, m) for m in MASKS_B]
        send0_a = (1 - bits_a[0]) * 8
        send0_b = (1 - bits_b[0]) * 8

        NB = 4
        RB = SQ // NB
        qb = lax.broadcasted_iota(jnp.int32, (SQ, SKV), 0) // 64
        kb = lax.broadcasted_iota(jnp.int32, (SQ, SKV), 1) // 64
        mask = kb <= qb
        band_masks = [mask[t * RB:(t + 1) * RB, :(t + 1) * RB]
                      for t in range(NB)]
        SC2 = 0.125 * 1.4426950408889634
        wq = wq_ref[...].astype(jnp.bfloat16)
        wo = wo_ref[...].astype(jnp.bfloat16)
        r0 = [None, None, None, None]
        for b in range(B):
            xb = x_ref[b].astype(jnp.bfloat16)
            q16 = jnp.dot(xb, wq,
                          preferred_element_type=jnp.float32).astype(jnp.bfloat16)
            band_parts = [[] for _ in range(NB)]
            for h in range(H_LOC):
                qh = q16[:, h * DH:(h + 1) * DH]
                kh = k_ref[b, :, h, :].astype(jnp.bfloat16)
                vh = v_ref[b, :, h, :].astype(jnp.bfloat16)
                for t in range(NB):
                    e = (t + 1) * RB
                    s = lax.dot_general(
                        qh[t * RB:(t + 1) * RB], kh[:e],
                        (((1,), (1,)), ((), ())),
                        preferred_element_type=jnp.float32) * SC2
                    w = jnp.exp2(jnp.where(band_masks[t], s, -1e9))
                    wsum = jnp.sum(w, axis=1, keepdims=True)
                    ctx_t = jnp.dot(w.astype(jnp.bfloat16), vh[:e],
                                    preferred_element_type=jnp.float32)
                    band_parts[t].append(ctx_t / wsum)
            ctx = jnp.concatenate(
                [jnp.concatenate(p, axis=1) for p in band_parts],
                axis=0).astype(jnp.bfloat16)
            partial = jnp.dot(ctx, wo, preferred_element_type=jnp.float32)
            p16 = partial.astype(jnp.bfloat16)
            for c in range(N_DEV):
                acc_a[_perm(MASKS_A, c), b] = p16[c * CH:(c + 1) * CH, :HDM]
                acc_b[_perm(MASKS_B, c), b] = p16[c * CH:(c + 1) * CH, HDM:]
            rd_a = pltpu.make_async_remote_copy(
                src_ref=acc_a.at[pl.ds(send0_a, 8), b],
                dst_ref=st_a.at[pl.ds(0, 8), b],
                send_sem=rsa_s.at[2 * b], recv_sem=rsa_r.at[2 * b],
                device_id=(jnp.bitwise_xor(my_i, MASKS_A[0]),),
                device_id_type=pl.DeviceIdType.MESH,
            )
            rd_a.start()
            rd_b = pltpu.make_async_remote_copy(
                src_ref=acc_b.at[pl.ds(send0_b, 8), b],
                dst_ref=st_b.at[pl.ds(0, 8), b],
                send_sem=rsb_s.at[2 * b], recv_sem=rsb_r.at[2 * b],
                device_id=(jnp.bitwise_xor(my_i, MASKS_B[0]),),
                device_id_type=pl.DeviceIdType.MESH,
            )
            rd_b.start()
            r0[2 * b], r0[2 * b + 1] = rd_a, rd_b

        for rd in r0:
            rd.wait()
        S_a = bits_a[0] * 8
        S_b = bits_b[0] * 8
        acc_a[pl.ds(S_a, 8)] = (
            acc_a[pl.ds(S_a, 8)].astype(jnp.float32)
            + st_a[pl.ds(0, 8)].astype(jnp.float32)).astype(jnp.bfloat16)
        acc_b[pl.ds(S_b, 8)] = (
            acc_b[pl.ds(S_b, 8)].astype(jnp.float32)
            + st_b[pl.ds(0, 8)].astype(jnp.float32)).astype(jnp.bfloat16)

        o = 8
        _SEMK = {1: 1, 2: 3, 3: 4}
        for k in (1, 2, 3):
            if k == 1:
                src_a = S_a + (1 - bits_a[1]) * 4
                src_b = S_b + (1 - bits_b[1]) * 4
            else:
                src_a, src_b = S_a, S_b
            rd_a = pltpu.make_async_remote_copy(
                src_ref=acc_a.at[pl.ds(src_a, 4)],
                dst_ref=st_a.at[pl.ds(o, 4)],
                send_sem=rsa_s.at[_SEMK[k]], recv_sem=rsa_r.at[_SEMK[k]],
                device_id=(jnp.bitwise_xor(my_i, MASKS_A[k]),),
                device_id_type=pl.DeviceIdType.MESH,
            )
            rd_a.start()
            rd_b = pltpu.make_async_remote_copy(
                src_ref=acc_b.at[pl.ds(src_b, 4)],
                dst_ref=st_b.at[pl.ds(o, 4)],
                send_sem=rsb_s.at[_SEMK[k]], recv_sem=rsb_r.at[_SEMK[k]],
                device_id=(jnp.bitwise_xor(my_i, MASKS_B[k]),),
                device_id_type=pl.DeviceIdType.MESH,
            )
            rd_b.start()
            rd_a.wait()
            if k == 1:
                S_a = S_a + bits_a[1] * 4
            acc_a[pl.ds(S_a, 4)] = (
                acc_a[pl.ds(S_a, 4)].astype(jnp.float32)
                + st_a[pl.ds(o, 4)].astype(jnp.float32)).astype(jnp.bfloat16)
            rd_b.wait()
            if k == 1:
                S_b = S_b + bits_b[1] * 4
            acc_b[pl.ds(S_b, 4)] = (
                acc_b[pl.ds(S_b, 4)].astype(jnp.float32)
                + st_b[pl.ds(o, 4)].astype(jnp.float32)).astype(jnp.bfloat16)
            o += 4

        for k in range(2):
            g = 4 << k
            m_a = MASKS_A[1 - k]
            m_b = MASKS_B[1 - k]
            rd_a = pltpu.make_async_remote_copy(
                src_ref=acc_a.at[pl.ds(S_a, g)],
                dst_ref=acc_a.at[pl.ds(S_a, g)],
                send_sem=aga_s.at[k], recv_sem=aga_r.at[k],
                device_id=(jnp.bitwise_xor(my_i, m_a),),
                device_id_type=pl.DeviceIdType.MESH,
            )
            rd_a.start()
            rd_b = pltpu.make_async_remote_copy(
                src_ref=acc_b.at[pl.ds(S_b, g)],
                dst_ref=acc_b.at[pl.ds(S_b, g)],
                send_sem=agb_s.at[k], recv_sem=agb_r.at[k],
                device_id=(jnp.bitwise_xor(my_i, m_b),),
                device_id_type=pl.DeviceIdType.MESH,
            )
            rd_b.start()
            rd_a.wait()
            shift = 2 + k
            sib = jnp.left_shift(jnp.bitwise_xor(
                lax.shift_right_logical(S_a, shift), 1), shift)
            S_a = jnp.minimum(S_a, sib)
            rd_b.wait()
            sib = jnp.left_shift(jnp.bitwise_xor(
                lax.shift_right_logical(S_b, shift), 1), shift)
            S_b = jnp.minimum(S_b, sib)

        for c in range(N_DEV):
            sl = slice(c * CH, (c + 1) * CH)
            out_ref[:, sl, :HDM] = acc_a[_perm(MASKS_A, c)].astype(jnp.float32)
            out_ref[:, sl, HDM:] = acc_b[_perm(MASKS_B, c)].astype(jnp.float32)

    return pl.pallas_call(
        body,
        out_shape=jax.ShapeDtypeStruct((B, SQ, DM), jnp.float32),
        in_specs=[pl.BlockSpec(memory_space=pltpu.VMEM)] * 5,
        out_specs=pl.BlockSpec(memory_space=pltpu.VMEM),
        scratch_shapes=[
            pltpu.VMEM((N_DEV, B, CH, HDM), jnp.bfloat16),
            pltpu.VMEM((N_DEV, B, CH, HDM), jnp.bfloat16),
            pltpu.VMEM((20, B, CH, HDM), jnp.bfloat16),
            pltpu.VMEM((20, B, CH, HDM), jnp.bfloat16),
            pltpu.SemaphoreType.DMA((5,)),
            pltpu.SemaphoreType.DMA((5,)),
            pltpu.SemaphoreType.DMA((5,)),
            pltpu.SemaphoreType.DMA((5,)),
            pltpu.SemaphoreType.DMA((2,)),
            pltpu.SemaphoreType.DMA((2,)),
            pltpu.SemaphoreType.DMA((2,)),
            pltpu.SemaphoreType.DMA((2,)),
        ],
    )(x, Wq_l, K_ext, V_ext, Wo_l)


# device time: 63323 ns/iter; 1.0735x vs baseline; 1.0023x over previous
import jax
import jax.numpy as jnp
from jax import lax
from jax.experimental import pallas as pl
from jax.experimental.pallas import tpu as pltpu

N_DEV = 16
B, SQ, SKV, DM = 2, 512, 512, 768
DH = 64
H_LOC = 8
DLOC = H_LOC * DH
CH = SQ // N_DEV
HDM = DM // 2
HS = SQ // 2

MASKS_A = [1, 4, 2, 8]
MASKS_B = [4, 1, 8, 2]


def _parity(v: int) -> int:
    return bin(v).count("1") & 1


def _perm(masks, c):
    pos = 0
    for k, m in enumerate(masks):
        pos |= _parity(c & m) << (3 - k)
    return pos


def _tparity(v, m):
    p = jnp.bitwise_and(v, m)
    p = jnp.bitwise_xor(p, lax.shift_right_logical(p, 2))
    p = jnp.bitwise_xor(p, lax.shift_right_logical(p, 1))
    return jnp.bitwise_and(p, 1)


def kernel(x, Wq, K_ext, V_ext, Wo):
    my = lax.axis_index("i")
    Wq_l = lax.dynamic_slice(Wq, (0, my * DLOC), (DM, DLOC))
    Wo_l = lax.dynamic_slice(Wo, (my * DLOC, 0), (DLOC, DM))

    def body(x_ref, wq_ref, k_ref, v_ref, wo_ref, out_ref,
             acc_a, acc_b, st_a, st_b,
             rsa_s, rsa_r, rsb_s, rsb_r, aga_s, aga_r, agb_s, agb_r):
        my_i = lax.axis_index("i")
        bits_a = [_tparity(my_i, m) for m in MASKS_A]
        bits_b = [_tparity(my_i, m) for m in MASKS_B]
        send0_a = (1 - bits_a[0]) * 8
        send0_b = (1 - bits_b[0]) * 8

        NB = 4
        RB = SQ // NB
        qb = lax.broadcasted_iota(jnp.int32, (SQ, SKV), 0) // 64
        kb = lax.broadcasted_iota(jnp.int32, (SQ, SKV), 1) // 64
        mask = kb <= qb
        band_masks = [mask[t * RB:(t + 1) * RB, :(t + 1) * RB]
                      for t in range(NB)]
        SC2 = 0.125 * 1.4426950408889634
        wq = wq_ref[...].astype(jnp.bfloat16)
        wo = wo_ref[...].astype(jnp.bfloat16)
        r0 = [None, None, None, None]
        for b in range(B):
            xb = x_ref[b].astype(jnp.bfloat16)
            q16 = jnp.dot(xb, wq,
                          preferred_element_type=jnp.float32).astype(jnp.bfloat16)
            band_parts = [[] for _ in range(NB)]
            for h in range(H_LOC):
                qh = q16[:, h * DH:(h + 1) * DH]
                kh = k_ref[b, :, h, :].astype(jnp.bfloat16)
                vh = v_ref[b, :, h, :].astype(jnp.bfloat16)
                for t in range(NB):
                    e = (t + 1) * RB
                    s = lax.dot_general(
                        qh[t * RB:(t + 1) * RB], kh[:e],
                        (((1,), (1,)), ((), ())),
                        preferred_element_type=jnp.float32) * SC2
                    w = jnp.exp2(jnp.where(band_masks[t], s, -1e9))
                    wsum = jnp.sum(w, axis=1, keepdims=True)
                    ctx_t = jnp.dot(w.astype(jnp.bfloat16), vh[:e],
                                    preferred_element_type=jnp.float32)
                    band_parts[t].append(ctx_t / wsum)
            ctx = jnp.concatenate(
                [jnp.concatenate(p, axis=1) for p in band_parts],
                axis=0).astype(jnp.bfloat16)
            partial = jnp.dot(ctx, wo, preferred_element_type=jnp.float32)
            p16 = partial.astype(jnp.bfloat16)
            for c in range(N_DEV):
                acc_a[_perm(MASKS_A, c), b] = p16[c * CH:(c + 1) * CH, :HDM]
                acc_b[_perm(MASKS_B, c), b] = p16[c * CH:(c + 1) * CH, HDM:]
            rd_a = pltpu.make_async_remote_copy(
                src_ref=acc_a.at[pl.ds(send0_a, 8), b],
                dst_ref=st_a.at[pl.ds(0, 8), b],
                send_sem=rsa_s.at[2 * b], recv_sem=rsa_r.at[2 * b],
                device_id=(jnp.bitwise_xor(my_i, MASKS_A[0]),),
                device_id_type=pl.DeviceIdType.MESH,
            )
            rd_a.start()
            rd_b = pltpu.make_async_remote_copy(
                src_ref=acc_b.at[pl.ds(send0_b, 8), b],
                dst_ref=st_b.at[pl.ds(0, 8), b],
                send_sem=rsb_s.at[2 * b], recv_sem=rsb_r.at[2 * b],
                device_id=(jnp.bitwise_xor(my_i, MASKS_B[0]),),
                device_id_type=pl.DeviceIdType.MESH,
            )
            rd_b.start()
            r0[2 * b], r0[2 * b + 1] = rd_a, rd_b

        for rd in r0:
            rd.wait()
        S_a = bits_a[0] * 8
        S_b = bits_b[0] * 8
        acc_a[pl.ds(S_a, 8)] = (
            acc_a[pl.ds(S_a, 8)].astype(jnp.float32)
            + st_a[pl.ds(0, 8)].astype(jnp.float32)).astype(jnp.bfloat16)
        acc_b[pl.ds(S_b, 8)] = (
            acc_b[pl.ds(S_b, 8)].astype(jnp.float32)
            + st_b[pl.ds(0, 8)].astype(jnp.float32)).astype(jnp.bfloat16)

        o = 8
        _SEMK = {1: 1, 2: 3, 3: 4}
        for k in (1, 2, 3):
            if k == 1:
                src_a = S_a + (1 - bits_a[1]) * 4
                src_b = S_b + (1 - bits_b[1]) * 4
            else:
                src_a, src_b = S_a, S_b
            rd_a = pltpu.make_async_remote_copy(
                src_ref=acc_a.at[pl.ds(src_a, 4)],
                dst_ref=st_a.at[pl.ds(o, 4)],
                send_sem=rsa_s.at[_SEMK[k]], recv_sem=rsa_r.at[_SEMK[k]],
                device_id=(jnp.bitwise_xor(my_i, MASKS_A[k]),),
                device_id_type=pl.DeviceIdType.MESH,
            )
            rd_a.start()
            rd_b = pltpu.make_async_remote_copy(
                src_ref=acc_b.at[pl.ds(src_b, 4)],
                dst_ref=st_b.at[pl.ds(o, 4)],
                send_sem=rsb_s.at[_SEMK[k]], recv_sem=rsb_r.at[_SEMK[k]],
                device_id=(jnp.bitwise_xor(my_i, MASKS_B[k]),),
                device_id_type=pl.DeviceIdType.MESH,
            )
            rd_b.start()
            rd_a.wait()
            if k == 1:
                S_a = S_a + bits_a[1] * 4
            acc_a[pl.ds(S_a, 4)] = (
                acc_a[pl.ds(S_a, 4)].astype(jnp.float32)
                + st_a[pl.ds(o, 4)].astype(jnp.float32)).astype(jnp.bfloat16)
            rd_b.wait()
            if k == 1:
                S_b = S_b + bits_b[1] * 4
            acc_b[pl.ds(S_b, 4)] = (
                acc_b[pl.ds(S_b, 4)].astype(jnp.float32)
                + st_b[pl.ds(o, 4)].astype(jnp.float32)).astype(jnp.bfloat16)
            o += 4

        def _bit(v, i):
            return jnp.bitwise_and(lax.shift_right_logical(v, i), 1)

        def _store_a(pos):
            c = (_bit(pos, 3)
                 | jnp.left_shift(_bit(pos, 2), 2)
                 | jnp.left_shift(_bit(pos, 1), 1)
                 | jnp.left_shift(jnp.bitwise_and(pos, 1), 3))
            out_ref[:, pl.ds(c * CH, CH), :HDM] = \
                acc_a[pl.ds(pos, 1)][0].astype(jnp.float32)

        def _store_b(pos):
            c = (jnp.left_shift(_bit(pos, 3), 2)
                 | _bit(pos, 2)
                 | jnp.left_shift(_bit(pos, 1), 3)
                 | jnp.left_shift(jnp.bitwise_and(pos, 1), 1))
            out_ref[:, pl.ds(c * CH, CH), HDM:] = \
                acc_b[pl.ds(pos, 1)][0].astype(jnp.float32)

        rd_a = pltpu.make_async_remote_copy(
            src_ref=acc_a.at[pl.ds(S_a, 4)], dst_ref=acc_a.at[pl.ds(S_a, 4)],
            send_sem=aga_s.at[0], recv_sem=aga_r.at[0],
            device_id=(jnp.bitwise_xor(my_i, MASKS_A[1]),),
            device_id_type=pl.DeviceIdType.MESH,
        )
        rd_a.start()
        rd_b = pltpu.make_async_remote_copy(
            src_ref=acc_b.at[pl.ds(S_b, 4)], dst_ref=acc_b.at[pl.ds(S_b, 4)],
            send_sem=agb_s.at[0], recv_sem=agb_r.at[0],
            device_id=(jnp.bitwise_xor(my_i, MASKS_B[1]),),
            device_id_type=pl.DeviceIdType.MESH,
        )
        rd_b.start()
        for j in range(4):
            _store_a(S_a + j)
            _store_b(S_b + j)
        rd_a.wait()
        rd_b.wait()
        sib4_a = jnp.left_shift(
            jnp.bitwise_xor(lax.shift_right_logical(S_a, 2), 1), 2)
        sib4_b = jnp.left_shift(
            jnp.bitwise_xor(lax.shift_right_logical(S_b, 2), 1), 2)
        W_a = jnp.minimum(S_a, sib4_a)
        W_b = jnp.minimum(S_b, sib4_b)

        rd_a = pltpu.make_async_remote_copy(
            src_ref=acc_a.at[pl.ds(W_a, 8)], dst_ref=acc_a.at[pl.ds(W_a, 8)],
            send_sem=aga_s.at[1], recv_sem=aga_r.at[1],
            device_id=(jnp.bitwise_xor(my_i, MASKS_A[0]),),
            device_id_type=pl.DeviceIdType.MESH,
        )
        rd_a.start()
        rd_b = pltpu.make_async_remote_copy(
            src_ref=acc_b.at[pl.ds(W_b, 8)], dst_ref=acc_b.at[pl.ds(W_b, 8)],
            send_sem=agb_s.at[1], recv_sem=agb_r.at[1],
            device_id=(jnp.bitwise_xor(my_i, MASKS_B[0]),),
            device_id_type=pl.DeviceIdType.MESH,
        )
        rd_b.start()
        for j in range(4):
            _store_a(sib4_a + j)
            _store_b(sib4_b + j)
        rd_a.wait()
        rd_b.wait()
        sib8_a = jnp.left_shift(
            jnp.bitwise_xor(lax.shift_right_logical(W_a, 3), 1), 3)
        sib8_b = jnp.left_shift(
            jnp.bitwise_xor(lax.shift_right_logical(W_b, 3), 1), 3)
        for j in range(8):
            _store_a(sib8_a + j)
            _store_b(sib8_b + j)

    return pl.pallas_call(
        body,
        out_shape=jax.ShapeDtypeStruct((B, SQ, DM), jnp.float32),
        in_specs=[pl.BlockSpec(memory_space=pltpu.VMEM)] * 5,
        out_specs=pl.BlockSpec(memory_space=pltpu.VMEM),
        scratch_shapes=[
            pltpu.VMEM((N_DEV, B, CH, HDM), jnp.bfloat16),
            pltpu.VMEM((N_DEV, B, CH, HDM), jnp.bfloat16),
            pltpu.VMEM((20, B, CH, HDM), jnp.bfloat16),
            pltpu.VMEM((20, B, CH, HDM), jnp.bfloat16),
            pltpu.SemaphoreType.DMA((5,)),
            pltpu.SemaphoreType.DMA((5,)),
            pltpu.SemaphoreType.DMA((5,)),
            pltpu.SemaphoreType.DMA((5,)),
            pltpu.SemaphoreType.DMA((2,)),
            pltpu.SemaphoreType.DMA((2,)),
            pltpu.SemaphoreType.DMA((2,)),
            pltpu.SemaphoreType.DMA((2,)),
        ],
    )(x, Wq_l, K_ext, V_ext, Wo_l)
